# baseline (speedup 1.0000x reference)
import functools

import jax
import jax.numpy as jnp
from jax import lax
from jax.experimental import pallas as pl
from jax.experimental.pallas import tpu as pltpu
from jax.experimental.pallas import tpu_sc as plsc

DIM_H = 1024
NV = 320
GR = 2
DCODE = 128
TB = 1024
LANES = 16


def _tc_body(hs_ref, w0_ref, w1_ref, b0_ref, b1_ref, idx_ref):
    hs = hs_ref[...]
    l0 = jnp.dot(hs, w0_ref[...], preferred_element_type=jnp.float32) + b0_ref[...]
    l1 = jnp.dot(hs, w1_ref[...], preferred_element_type=jnp.float32) + b1_ref[...]
    iota = lax.broadcasted_iota(jnp.int32, (TB, NV), 1)
    big = jnp.int32(2 ** 30)

    m0 = jnp.max(l0, axis=1, keepdims=True)
    i0 = jnp.min(jnp.where(l0 == m0, iota, big), axis=1)
    m1 = jnp.max(l1, axis=1, keepdims=True)
    i1 = jnp.min(jnp.where(l1 == m1, iota, big), axis=1)

    idx_ref[:, 0] = i0
    idx_ref[:, 1] = i1 + NV


def _tc_call(hs2d, w0, w1, b0, b1):
    T = hs2d.shape[0]
    return pl.pallas_call(
        _tc_body,
        grid=(T // TB,),
        in_specs=[
            pl.BlockSpec((TB, DIM_H), lambda i: (i, 0)),
            pl.BlockSpec((DIM_H, NV), lambda i: (0, 0)),
            pl.BlockSpec((DIM_H, NV), lambda i: (0, 0)),
            pl.BlockSpec((1, NV), lambda i: (0, 0)),
            pl.BlockSpec((1, NV), lambda i: (0, 0)),
        ],
        out_specs=pl.BlockSpec((TB, GR), lambda i: (i, 0)),
        out_shape=jax.ShapeDtypeStruct((T, GR), jnp.int32),
    )(hs2d, w0, w1, b0, b1)


@functools.lru_cache(maxsize=None)
def _make_sc_dist_gather(B):
    info = plsc.get_sparse_core_info()
    nw = info.num_cores * info.num_subcores
    rows_w = B // nw
    mesh = plsc.VectorSubcoreMesh(core_axis_name="c", subcore_axis_name="s")

    @functools.partial(
        pl.kernel,
        mesh=mesh,
        out_type=[
            jax.ShapeDtypeStruct((B * NV,), jnp.float32),
            jax.ShapeDtypeStruct((B, DCODE), jnp.float32),
        ],
        scratch_types=[
            pltpu.VMEM((rows_w,), jnp.int32),
            pltpu.VMEM((rows_w * NV,), jnp.float32),
            pltpu.VMEM((rows_w, DCODE), jnp.float32),
            pltpu.SemaphoreType.DMA,
        ],
    )
    def k(table_hbm, idx_hbm, dist_hbm, cv_hbm, idx_v, db_v, rows_v, sem):
        wid = lax.axis_index("s") * info.num_cores + lax.axis_index("c")
        base = wid * rows_w
        pltpu.sync_copy(idx_hbm.at[pl.ds(base, rows_w)], idx_v)
        gather = pltpu.async_copy(table_hbm.at[idx_v], rows_v, sem)

        zeros = jnp.zeros((LANES,), jnp.float32)

        def zero_row(i, _):
            for kk in range(NV // LANES):
                db_v[pl.ds(i * NV + kk * LANES, LANES)] = zeros
            return _

        lax.fori_loop(0, rows_w, zero_row, 0)

        lane = lax.iota(jnp.int32, LANES)

        def put_row16(j, _):
            vj = idx_v[pl.ds(j * LANES, LANES)]
            for l in range(LANES):
                col = vj[l] - NV * (l & 1)
                blk = col & jnp.int32(-LANES)
                vec = jnp.where(lane == col - blk, jnp.float32(1.0),
                                jnp.float32(0.0))
                off = (j * LANES + l) * NV + blk
                db_v[pl.ds(off, LANES)] = vec
            return _

        lax.fori_loop(0, rows_w // LANES, put_row16, 0)

        pltpu.sync_copy(db_v, dist_hbm.at[pl.ds(base * NV, rows_w * NV)])
        gather.wait()
        pltpu.sync_copy(rows_v, cv_hbm.at[pl.ds(base, rows_w)])

    return k


def kernel(hidden_states, codevectors, w_proj, b_proj):
    B, S, H = hidden_states.shape
    T = B * S
    hs2d = hidden_states.reshape(T, H)
    w0 = w_proj[:, :NV]
    w1 = w_proj[:, NV:]
    b0 = b_proj[:NV].reshape(1, NV)
    b1 = b_proj[NV:].reshape(1, NV)
    idx = _tc_call(hs2d, w0, w1, b0, b1)

    table = codevectors.reshape(GR * NV, DCODE)
    flat_idx = idx.reshape(T * GR)
    dist_flat, cv = _make_sc_dist_gather(T * GR)(table, flat_idx)
    dist = dist_flat.reshape(T, GR, NV)

    cv = cv.reshape(B, S, GR * DCODE)
    return cv, dist


# speedup vs baseline: 1.2783x; 1.2783x over previous
import functools

import jax
import jax.numpy as jnp
from jax import lax
from jax.experimental import pallas as pl
from jax.experimental.pallas import tpu as pltpu
from jax.experimental.pallas import tpu_sc as plsc

DIM_H = 1024
NV = 320
GR = 2
DCODE = 128
TB = 1024


def _argmax_body(hs_ref, w0_ref, w1_ref, b0_ref, b1_ref, idx_ref):
    hs = hs_ref[...]
    l0 = jnp.dot(hs, w0_ref[...], preferred_element_type=jnp.float32) + b0_ref[...]
    l1 = jnp.dot(hs, w1_ref[...], preferred_element_type=jnp.float32) + b1_ref[...]
    iota = lax.broadcasted_iota(jnp.int32, (TB, NV), 1)
    big = jnp.int32(2 ** 30)

    m0 = jnp.max(l0, axis=1, keepdims=True)
    i0 = jnp.min(jnp.where(l0 == m0, iota, big), axis=1)
    m1 = jnp.max(l1, axis=1, keepdims=True)
    i1 = jnp.min(jnp.where(l1 == m1, iota, big), axis=1)

    idx_ref[:, 0] = i0
    idx_ref[:, 1] = i1 + NV


def _argmax_call(hs2d, w0, w1, b0, b1):
    T = hs2d.shape[0]
    return pl.pallas_call(
        _argmax_body,
        grid=(T // TB,),
        in_specs=[
            pl.BlockSpec((TB, DIM_H), lambda i: (i, 0)),
            pl.BlockSpec((DIM_H, NV), lambda i: (0, 0)),
            pl.BlockSpec((DIM_H, NV), lambda i: (0, 0)),
            pl.BlockSpec((1, NV), lambda i: (0, 0)),
            pl.BlockSpec((1, NV), lambda i: (0, 0)),
        ],
        out_specs=pl.BlockSpec((TB, GR), lambda i: (i, 0)),
        out_shape=jax.ShapeDtypeStruct((T, GR), jnp.int32),
    )(hs2d, w0, w1, b0, b1)


def _dist_body(idx_ref, dist_ref):
    iota = lax.broadcasted_iota(jnp.int32, (TB, NV), 1)
    i0 = idx_ref[:, 0]
    i1 = idx_ref[:, 1] - NV
    dist_ref[:, 0, :] = (iota == i0[:, None]).astype(jnp.float32)
    dist_ref[:, 1, :] = (iota == i1[:, None]).astype(jnp.float32)


def _dist_call(idx):
    T = idx.shape[0]
    return pl.pallas_call(
        _dist_body,
        grid=(T // TB,),
        in_specs=[pl.BlockSpec((TB, GR), lambda i: (i, 0))],
        out_specs=pl.BlockSpec((TB, GR, NV), lambda i: (i, 0, 0)),
        out_shape=jax.ShapeDtypeStruct((T, GR, NV), jnp.float32),
    )(idx)


@functools.lru_cache(maxsize=None)
def _make_sc_gather(B, D):
    info = plsc.get_sparse_core_info()
    nw = info.num_cores * info.num_subcores
    b_per_w = B // nw
    mesh = plsc.VectorSubcoreMesh(core_axis_name="c", subcore_axis_name="s")

    @functools.partial(
        pl.kernel,
        mesh=mesh,
        out_type=jax.ShapeDtypeStruct((B, D), jnp.float32),
        scratch_types=[
            pltpu.VMEM((b_per_w,), jnp.int32),
            pltpu.VMEM((b_per_w, D), jnp.float32),
            pltpu.SemaphoreType.DMA,
        ],
    )
    def k(table_hbm, idx_hbm, out_hbm, idx_v, rows_v, sem):
        wid = lax.axis_index("s") * info.num_cores + lax.axis_index("c")
        base = wid * b_per_w
        pltpu.sync_copy(idx_hbm.at[pl.ds(base, b_per_w)], idx_v)
        pltpu.async_copy(table_hbm.at[idx_v], rows_v, sem).wait()
        pltpu.sync_copy(rows_v, out_hbm.at[pl.ds(base, b_per_w)])

    return k


def kernel(hidden_states, codevectors, w_proj, b_proj):
    B, S, H = hidden_states.shape
    T = B * S
    hs2d = hidden_states.reshape(T, H)
    w0 = w_proj[:, :NV]
    w1 = w_proj[:, NV:]
    b0 = b_proj[:NV].reshape(1, NV)
    b1 = b_proj[NV:].reshape(1, NV)
    idx = _argmax_call(hs2d, w0, w1, b0, b1)

    table = codevectors.reshape(GR * NV, DCODE)
    flat_idx = idx.reshape(T * GR)
    cv = _make_sc_gather(T * GR, DCODE)(table, flat_idx)
    cv = cv.reshape(B, S, GR * DCODE)
    dist = _dist_call(idx)
    return cv, dist


# speedup vs baseline: 1.8992x; 1.4857x over previous
import functools

import jax
import jax.numpy as jnp
from jax import lax
from jax.experimental import pallas as pl
from jax.experimental.pallas import tpu as pltpu
from jax.experimental.pallas import tpu_sc as plsc

DIM_H = 1024
NV = 320
GR = 2
DCODE = 128
TB = 1024


def _argmax_body(hs_ref, w_ref, b_ref, idx0_ref, idx1_ref):
    hs = hs_ref[...]
    l = jnp.dot(hs, w_ref[...], preferred_element_type=jnp.float32)
    l = l + b_ref[...][None, :]
    iota = lax.broadcasted_iota(jnp.int32, (TB, NV), 1)
    big = jnp.int32(2 ** 30)

    l0 = l[:, :NV]
    l1 = l[:, NV:]
    m0 = jnp.max(l0, axis=1, keepdims=True)
    i0 = jnp.min(jnp.where(l0 == m0, iota, big), axis=1)
    m1 = jnp.max(l1, axis=1, keepdims=True)
    i1 = jnp.min(jnp.where(l1 == m1, iota, big), axis=1)

    idx0_ref[...] = i0
    idx1_ref[...] = i1 + NV


def _argmax_call(hs2d, w_proj, b_proj):
    T = hs2d.shape[0]
    return pl.pallas_call(
        _argmax_body,
        grid=(T // TB,),
        in_specs=[
            pl.BlockSpec((TB, DIM_H), lambda i: (i, 0)),
            pl.BlockSpec((DIM_H, GR * NV), lambda i: (0, 0)),
            pl.BlockSpec((GR * NV,), lambda i: (0,)),
        ],
        out_specs=[
            pl.BlockSpec((TB,), lambda i: (i,)),
            pl.BlockSpec((TB,), lambda i: (i,)),
        ],
        out_shape=[
            jax.ShapeDtypeStruct((T,), jnp.int32),
            jax.ShapeDtypeStruct((T,), jnp.int32),
        ],
    )(hs2d, w_proj, b_proj)


def _dist_body(idx0_ref, idx1_ref, dist_ref):
    iota = lax.broadcasted_iota(jnp.int32, (NV, TB), 0)
    i0 = idx0_ref[...]
    i1 = idx1_ref[...] - NV
    dist_ref[0] = (iota == i0[None, :]).astype(jnp.float32)
    dist_ref[1] = (iota == i1[None, :]).astype(jnp.float32)


def _dist_call(idx0, idx1):
    T = idx0.shape[0]
    return pl.pallas_call(
        _dist_body,
        grid=(T // TB,),
        in_specs=[
            pl.BlockSpec((TB,), lambda i: (i,)),
            pl.BlockSpec((TB,), lambda i: (i,)),
        ],
        out_specs=pl.BlockSpec((GR, NV, TB), lambda i: (0, 0, i)),
        out_shape=jax.ShapeDtypeStruct((GR, NV, T), jnp.float32),
    )(idx0, idx1)


@functools.lru_cache(maxsize=None)
def _make_sc_gather(T):
    info = plsc.get_sparse_core_info()
    nw = info.num_cores * info.num_subcores
    t_per_w = T // nw
    mesh = plsc.VectorSubcoreMesh(core_axis_name="c", subcore_axis_name="s")

    @functools.partial(
        pl.kernel,
        mesh=mesh,
        out_type=jax.ShapeDtypeStruct((T, GR * DCODE), jnp.float32),
        scratch_types=[
            pltpu.VMEM((t_per_w,), jnp.int32),
            pltpu.VMEM((t_per_w,), jnp.int32),
            pltpu.VMEM((t_per_w, DCODE), jnp.float32),
            pltpu.VMEM((t_per_w, DCODE), jnp.float32),
            pltpu.SemaphoreType.DMA,
            pltpu.SemaphoreType.DMA,
        ],
    )
    def k(table_hbm, idx0_hbm, idx1_hbm, out_hbm, ia_v, ib_v, g0_v, g1_v,
          sem0, sem1):
        wid = lax.axis_index("s") * info.num_cores + lax.axis_index("c")
        base = wid * t_per_w
        pltpu.sync_copy(idx0_hbm.at[pl.ds(base, t_per_w)], ia_v)
        pltpu.sync_copy(idx1_hbm.at[pl.ds(base, t_per_w)], ib_v)
        c0 = pltpu.async_copy(table_hbm.at[ia_v], g0_v, sem0)
        c1 = pltpu.async_copy(table_hbm.at[ib_v], g1_v, sem1)
        c0.wait()
        c1.wait()
        pltpu.sync_copy(g0_v, out_hbm.at[pl.ds(base, t_per_w), pl.ds(0, DCODE)])
        pltpu.sync_copy(g1_v, out_hbm.at[pl.ds(base, t_per_w), pl.ds(DCODE, DCODE)])

    return k


def kernel(hidden_states, codevectors, w_proj, b_proj):
    B, S, H = hidden_states.shape
    T = B * S
    hs2d = hidden_states.reshape(T, H)
    idx0, idx1 = _argmax_call(hs2d, w_proj, b_proj)

    table = codevectors.reshape(GR * NV, DCODE)
    cv = _make_sc_gather(T)(table, idx0, idx1)
    cv = cv.reshape(B, S, GR * DCODE)
    dist_t = _dist_call(idx0, idx1)
    dist = jnp.transpose(dist_t, (2, 0, 1))
    return cv, dist
